# baseline (speedup 1.0000x reference)
import jax
import jax.numpy as jnp
from jax.experimental import pallas as pl
from jax.experimental.pallas import tpu as pltpu

B, C, H, W = 16, 768, 32, 32
HW = H * W
K = 4
P = 8
CH = 256
DH = 3
RO = 2
NEG_INF = float("-inf")


def _body(feat_ref, w_ref, z_ref, peaks_ref):
    wscal = w_ref[0, 0, 0, 0]

    cols = jax.lax.broadcasted_iota(jnp.int32, (1, HW), 1)
    hh = cols // W
    ww = cols % W
    cols_p = jax.lax.broadcasted_iota(jnp.int32, (1, 2 * K), 1)

    sal_rows = []
    for p in range(P):
        fm = feat_ref[p]

        cm_acc = jnp.zeros((C, CH), jnp.float32)
        pm_parts = []
        for j in range(HW // CH):
            x = fm[:, j * CH:(j + 1) * CH]
            cm_acc = cm_acc + x
            pm_parts.append(jnp.sum(x, axis=0, keepdims=True))
        cm = jnp.sum(cm_acc, axis=1, keepdims=True)
        pm = jnp.concatenate(pm_parts, axis=1)

        cme = jnp.exp(cm * jnp.float32(1.0 / HW))
        wsum = jnp.sum(fm * cme, axis=0, keepdims=True)
        pme = jnp.exp(pm * (jnp.float32(1.0 / C) * wscal))
        sal_rows.append(wsum * pme)

    sal = jnp.concatenate(sal_rows, axis=0)

    wins = []
    pv = jnp.zeros((P, 2 * K), jnp.int32)
    for k in range(K):
        mx = jnp.max(sal, axis=1, keepdims=True)
        idx = jnp.min(jnp.where(sal == mx, cols, HW), axis=1, keepdims=True)
        ph = idx // W
        pw = idx % W
        pv = pv + jnp.where(cols_p == 2 * k, ph, 0) \
                + jnp.where(cols_p == 2 * k + 1, pw, 0)
        dh = jnp.abs(hh - ph)
        dw = jnp.abs(ww - pw)
        sal = jnp.where((dh <= DH) & (dw <= DH), NEG_INF, sal)
        nh = jnp.minimum(ph + RO, H - 1) - jnp.maximum(ph - RO, 0) + 1
        nw = jnp.minimum(pw + RO, W - 1) - jnp.maximum(pw - RO, 0) + 1
        inv = jnp.float32(1.0) / (nh * nw).astype(jnp.float32)
        wins.append(((dh <= RO) & (dw <= RO)).astype(jnp.float32) * inv)

    peaks_ref[...] = pv.reshape(P, 1, 2 * K)

    for p in range(P):
        wmap = jnp.concatenate([wk[p:p + 1] for wk in wins], axis=0)
        z_ref[p] = jax.lax.dot_general(
            wmap, feat_ref[p], (((1,), (1,)), ((), ())),
            preferred_element_type=jnp.float32)


@jax.jit
def kernel(feat, w):
    z, peaks = pl.pallas_call(
        _body,
        grid=(B // P,),
        in_specs=[
            pl.BlockSpec((P, C, HW), lambda b: (b, 0, 0)),
            pl.BlockSpec((1, 1, 1, 1), lambda b: (0, 0, 0, 0)),
        ],
        out_specs=[
            pl.BlockSpec((P, K, C), lambda b: (b, 0, 0)),
            pl.BlockSpec((P, 1, 2 * K), lambda b: (b, 0, 0)),
        ],
        out_shape=[
            jax.ShapeDtypeStruct((B, K, C), jnp.float32),
            jax.ShapeDtypeStruct((B, 1, 2 * K), jnp.int32),
        ],
        compiler_params=pltpu.CompilerParams(
            dimension_semantics=("arbitrary",)),
    )(feat.reshape(B, C, HW), w)
    return z, peaks.reshape(B, K, 2)


# speedup vs baseline: 1.0538x; 1.0538x over previous
import jax
import jax.numpy as jnp
from jax.experimental import pallas as pl
from jax.experimental.pallas import tpu as pltpu

B, C, H, W = 16, 768, 32, 32
HW = H * W
K = 4
P = 4
CH = 256
DH = 3
RO = 2
NEG_INF = float("-inf")


def _body(feat_ref, w_ref, z_ref, peaks_ref):
    wscal = w_ref[0, 0, 0, 0]

    cols = jax.lax.broadcasted_iota(jnp.int32, (1, HW), 1)
    hh = cols // W
    ww = cols % W
    cols_p = jax.lax.broadcasted_iota(jnp.int32, (1, 2 * K), 1)

    sal_rows = []
    for p in range(P):
        fm = feat_ref[p]

        cm_acc = jnp.zeros((C, CH), jnp.float32)
        pm_parts = []
        for j in range(HW // CH):
            x = fm[:, j * CH:(j + 1) * CH]
            cm_acc = cm_acc + x
            pm_parts.append(jnp.sum(x, axis=0, keepdims=True))
        cm = jnp.sum(cm_acc, axis=1, keepdims=True)
        pm = jnp.concatenate(pm_parts, axis=1)

        cme = jnp.exp(cm * jnp.float32(1.0 / HW))
        wsum = jnp.sum(fm * cme, axis=0, keepdims=True)
        pme = jnp.exp(pm * (jnp.float32(1.0 / C) * wscal))
        sal_rows.append(wsum * pme)

    sal = jnp.concatenate(sal_rows, axis=0)

    wins = []
    pv = jnp.zeros((P, 2 * K), jnp.int32)
    for k in range(K):
        mx = jnp.max(sal, axis=1, keepdims=True)
        idx = jnp.min(jnp.where(sal == mx, cols, HW), axis=1, keepdims=True)
        ph = idx // W
        pw = idx % W
        pv = pv + jnp.where(cols_p == 2 * k, ph, 0) \
                + jnp.where(cols_p == 2 * k + 1, pw, 0)
        dh = jnp.abs(hh - ph)
        dw = jnp.abs(ww - pw)
        sal = jnp.where((dh <= DH) & (dw <= DH), NEG_INF, sal)
        nh = jnp.minimum(ph + RO, H - 1) - jnp.maximum(ph - RO, 0) + 1
        nw = jnp.minimum(pw + RO, W - 1) - jnp.maximum(pw - RO, 0) + 1
        inv = jnp.float32(1.0) / (nh * nw).astype(jnp.float32)
        wins.append(((dh <= RO) & (dw <= RO)).astype(jnp.float32) * inv)

    peaks_ref[...] = pv.reshape(P, 1, 2 * K)

    for p in range(P):
        wmap = jnp.concatenate([wk[p:p + 1] for wk in wins], axis=0)
        z_ref[p] = jax.lax.dot_general(
            wmap, feat_ref[p], (((1,), (1,)), ((), ())),
            preferred_element_type=jnp.float32)


@jax.jit
def kernel(feat, w):
    z, peaks = pl.pallas_call(
        _body,
        grid=(B // P,),
        in_specs=[
            pl.BlockSpec((P, C, HW), lambda b: (b, 0, 0)),
            pl.BlockSpec((1, 1, 1, 1), lambda b: (0, 0, 0, 0)),
        ],
        out_specs=[
            pl.BlockSpec((P, K, C), lambda b: (b, 0, 0)),
            pl.BlockSpec((P, 1, 2 * K), lambda b: (b, 0, 0)),
        ],
        out_shape=[
            jax.ShapeDtypeStruct((B, K, C), jnp.float32),
            jax.ShapeDtypeStruct((B, 1, 2 * K), jnp.int32),
        ],
        compiler_params=pltpu.CompilerParams(
            dimension_semantics=("parallel",)),
    )(feat.reshape(B, C, HW), w)
    return z, peaks.reshape(B, K, 2)
